# Initial kernel scaffold; baseline (speedup 1.0000x reference)
import jax
import jax.numpy as jnp
from jax.experimental import pallas as pl


def kernel(graph1_x, graph2_x, idx_left, idx_right, pair_seg, g1_len, g2_len, W1, b1, W2, b2):
    raise NotImplementedError("write your pallas kernel here")


# speedup vs baseline: 4.4660x; 4.4660x over previous
import functools

import jax
import jax.numpy as jnp
from jax import lax
from jax.experimental import pallas as pl
from jax.experimental.pallas import tpu as pltpu
from jax.experimental.pallas import tpu_sc as plsc

_NC = 2
_NS = 16
_NW = _NC * _NS
_CH = 128
_L = 16


def _proj_body(g1_ref, g2_ref, w1a_ref, w1b_ref, b1_ref, a_ref, c_ref):
    a_ref[...] = jnp.dot(g1_ref[...], w1a_ref[...],
                         preferred_element_type=jnp.float32)
    c_ref[...] = jnp.dot(g2_ref[...], w1b_ref[...],
                         preferred_element_type=jnp.float32) + b1_ref[...]


def _idx_body(lenl_ref, lenr_ref, idxl_ref, idxr_ref, seg_ref, gl_ref, gr_ref):
    seg = seg_ref[...]
    offl = jnp.zeros_like(seg)
    offr = jnp.zeros_like(seg)
    runl = jnp.int32(0)
    runr = jnp.int32(0)
    nseg = lenl_ref.shape[0]
    for s in range(nseg):
        offl = offl + jnp.where(seg == s, runl, 0)
        offr = offr + jnp.where(seg == s, runr, 0)
        runl = runl + lenl_ref[s]
        runr = runr + lenr_ref[s]
    gl_ref[...] = idxl_ref[...] + offl
    gr_ref[...] = idxr_ref[...] + offr


def _sc_body(nchunk, ppw, a_hbm, c_hbm, gl_hbm, gr_hbm, w2_hbm, b2_hbm,
             out_hbm, gl_v, gr_v, rowsa_v, rowsc_v, outbuf_v,
             w2_v, b2_v, sem_a, sem_c):
    cid = lax.axis_index("c")
    sid = lax.axis_index("s")
    wid = sid * _NC + cid
    base_w = wid * ppw

    pltpu.sync_copy(w2_hbm, w2_v)
    pltpu.sync_copy(b2_hbm, b2_v)
    b2vec = b2_v[...]
    w2regs = [w2_v[pl.ds(j * _L, _L)] for j in range(8)]
    lane = lax.iota(jnp.int32, _L)

    def chunk_body(ch, carry):
        base = base_w + ch * _CH
        pltpu.sync_copy(gl_hbm.at[pl.ds(base, _CH)], gl_v)
        pltpu.sync_copy(gr_hbm.at[pl.ds(base, _CH)], gr_v)
        cpa = pltpu.async_copy(a_hbm.at[gl_v], rowsa_v, sem_a)
        cpc = pltpu.async_copy(c_hbm.at[gr_v], rowsc_v, sem_c)
        cpa.wait()
        cpc.wait()

        def group_body(g, gcarry):
            ovec = b2vec
            for i in range(_L):
                p = g * _L + i
                acc = jnp.zeros((_L,), jnp.float32)
                for j in range(8):
                    va = rowsa_v[p, pl.ds(j * _L, _L)]
                    vc = rowsc_v[p, pl.ds(j * _L, _L)]
                    acc = acc + jnp.maximum(va + vc, 0.0) * w2regs[j]
                ovec = jnp.where(lane == i, ovec + jnp.sum(acc), ovec)
            outbuf_v[pl.ds(g * _L, _L)] = ovec
            return gcarry

        lax.fori_loop(0, _CH // _L, group_body, 0)
        pltpu.sync_copy(outbuf_v, out_hbm.at[pl.ds(base, _CH)])
        return carry

    lax.fori_loop(0, nchunk, chunk_body, 0)


def kernel(graph1_x, graph2_x, idx_left, idx_right, pair_seg, g1_len, g2_len,
           W1, b1, W2, b2):
    n, d = graph1_x.shape
    ed = W1.shape[0]
    p = idx_left.shape[0]
    nseg = g1_len.shape[0]

    row_blk = 2048
    proj = pl.pallas_call(
        _proj_body,
        grid=(n // row_blk,),
        in_specs=[
            pl.BlockSpec((row_blk, d), lambda i: (i, 0)),
            pl.BlockSpec((row_blk, d), lambda i: (i, 0)),
            pl.BlockSpec((d, ed), lambda i: (0, 0)),
            pl.BlockSpec((d, ed), lambda i: (0, 0)),
            pl.BlockSpec((1, ed), lambda i: (0, 0)),
        ],
        out_specs=[
            pl.BlockSpec((row_blk, ed), lambda i: (i, 0)),
            pl.BlockSpec((row_blk, ed), lambda i: (i, 0)),
        ],
        out_shape=[
            jax.ShapeDtypeStruct((n, ed), jnp.float32),
            jax.ShapeDtypeStruct((n, ed), jnp.float32),
        ],
    )
    a_t, c_t = proj(graph1_x, graph2_x, W1[:d], W1[d:], b1.reshape(1, ed))

    pc = 128
    pr = p // pc
    blk_r = 128
    idx_call = pl.pallas_call(
        _idx_body,
        grid=(pr // blk_r,),
        in_specs=[
            pl.BlockSpec(memory_space=pltpu.SMEM),
            pl.BlockSpec(memory_space=pltpu.SMEM),
            pl.BlockSpec((blk_r, pc), lambda i: (i, 0)),
            pl.BlockSpec((blk_r, pc), lambda i: (i, 0)),
            pl.BlockSpec((blk_r, pc), lambda i: (i, 0)),
        ],
        out_specs=[
            pl.BlockSpec((blk_r, pc), lambda i: (i, 0)),
            pl.BlockSpec((blk_r, pc), lambda i: (i, 0)),
        ],
        out_shape=[
            jax.ShapeDtypeStruct((pr, pc), jnp.int32),
            jax.ShapeDtypeStruct((pr, pc), jnp.int32),
        ],
    )
    gl2, gr2 = idx_call(g1_len, g2_len, idx_left.reshape(pr, pc),
                        idx_right.reshape(pr, pc), pair_seg.reshape(pr, pc))
    gl = gl2.reshape(p)
    gr = gr2.reshape(p)

    ppw = p // _NW
    nchunk = ppw // _CH
    mesh = plsc.VectorSubcoreMesh(core_axis_name="c", subcore_axis_name="s")
    sc_call = pl.kernel(
        functools.partial(_sc_body, nchunk, ppw),
        out_type=jax.ShapeDtypeStruct((p,), jnp.float32),
        mesh=mesh,
        compiler_params=pltpu.CompilerParams(needs_layout_passes=False),
        scratch_types=[
            pltpu.VMEM((_CH,), jnp.int32),
            pltpu.VMEM((_CH,), jnp.int32),
            pltpu.VMEM((_CH, ed), jnp.float32),
            pltpu.VMEM((_CH, ed), jnp.float32),
            pltpu.VMEM((_CH,), jnp.float32),
            pltpu.VMEM((ed,), jnp.float32),
            pltpu.VMEM((_L,), jnp.float32),
            pltpu.SemaphoreType.DMA,
            pltpu.SemaphoreType.DMA,
        ],
    )
    b2vec = jnp.full((_L,), b2[0], dtype=jnp.float32)
    out = sc_call(a_t, c_t, gl, gr, W2.reshape(ed), b2vec)
    return out.reshape(p, 1)


# speedup vs baseline: 5.9187x; 1.3253x over previous
import functools

import jax
import jax.numpy as jnp
from jax import lax
from jax.experimental import pallas as pl
from jax.experimental.pallas import tpu as pltpu
from jax.experimental.pallas import tpu_sc as plsc

_NC = 2
_NS = 16
_NW = _NC * _NS
_CH = 128
_L = 16


def _prep_body(lenl_ref, lenr_ref, g1_ref, g2_ref, w1a_ref, w1b_ref, b1_ref,
               idxl_ref, idxr_ref, seg_ref, a_ref, c_ref, gl_ref, gr_ref):
    a_ref[...] = jnp.dot(g1_ref[...], w1a_ref[...],
                         preferred_element_type=jnp.float32)
    c_ref[...] = jnp.dot(g2_ref[...], w1b_ref[...],
                         preferred_element_type=jnp.float32) + b1_ref[...]
    seg = seg_ref[...]
    offl = jnp.zeros_like(seg)
    offr = jnp.zeros_like(seg)
    runl = jnp.int32(0)
    runr = jnp.int32(0)
    nseg = lenl_ref.shape[0]
    for s in range(nseg):
        offl = offl + jnp.where(seg == s, runl, 0)
        offr = offr + jnp.where(seg == s, runr, 0)
        runl = runl + lenl_ref[s]
        runr = runr + lenr_ref[s]
    gl_ref[...] = idxl_ref[...] + offl
    gr_ref[...] = idxr_ref[...] + offr


def _sc_body(nchunk, ppw, a_hbm, c_hbm, gl_hbm, gr_hbm, w2_hbm, b2_hbm,
             out_hbm, gl0_v, gr0_v, ra0_v, rc0_v, gl1_v, gr1_v, ra1_v, rc1_v,
             outbuf_v, w2_v, b2_v, sa0, sc0, sa1, sc1):
    cid = lax.axis_index("c")
    sid = lax.axis_index("s")
    wid = sid * _NC + cid
    base_w = wid * ppw

    pltpu.sync_copy(w2_hbm, w2_v)
    pltpu.sync_copy(b2_hbm, b2_v)
    b2vec = b2_v[...]
    w2regs = [w2_v[pl.ds(j * _L, _L)] for j in range(8)]
    lane = lax.iota(jnp.int32, _L)

    bufs = ((gl0_v, gr0_v, ra0_v, rc0_v, sa0, sc0),
            (gl1_v, gr1_v, ra1_v, rc1_v, sa1, sc1))

    def issue(buf, base):
        gl_v, gr_v, ra_v, rc_v, sem_a, sem_c = buf
        pltpu.sync_copy(gl_hbm.at[pl.ds(base, _CH)], gl_v)
        pltpu.sync_copy(gr_hbm.at[pl.ds(base, _CH)], gr_v)
        pltpu.async_copy(a_hbm.at[gl_v], ra_v, sem_a)
        pltpu.async_copy(c_hbm.at[gr_v], rc_v, sem_c)

    def drain(buf):
        gl_v, gr_v, ra_v, rc_v, sem_a, sem_c = buf
        pltpu.make_async_copy(a_hbm.at[gl_v], ra_v, sem_a).wait()
        pltpu.make_async_copy(c_hbm.at[gr_v], rc_v, sem_c).wait()

    def compute(buf, base):
        gl_v, gr_v, ra_v, rc_v, sem_a, sem_c = buf

        def group_body(g, gcarry):
            ovec = b2vec
            for i in range(_L):
                p = g * _L + i
                acc = jnp.zeros((_L,), jnp.float32)
                for j in range(8):
                    va = ra_v[p, pl.ds(j * _L, _L)]
                    vc = rc_v[p, pl.ds(j * _L, _L)]
                    acc = acc + jnp.maximum(va + vc, 0.0) * w2regs[j]
                ovec = jnp.where(lane == i, ovec + jnp.sum(acc), ovec)
            outbuf_v[pl.ds(g * _L, _L)] = ovec
            return gcarry

        lax.fori_loop(0, _CH // _L, group_body, 0)
        pltpu.sync_copy(outbuf_v, out_hbm.at[pl.ds(base, _CH)])

    half = nchunk // 2
    issue(bufs[0], base_w)

    def body2(it, carry):
        base0 = base_w + (2 * it) * _CH
        drain(bufs[0])
        issue(bufs[1], base0 + _CH)
        compute(bufs[0], base0)
        drain(bufs[1])

        @pl.when(it < half - 1)
        def _():
            issue(bufs[0], base0 + 2 * _CH)

        compute(bufs[1], base0 + _CH)
        return carry

    lax.fori_loop(0, half, body2, 0)


def kernel(graph1_x, graph2_x, idx_left, idx_right, pair_seg, g1_len, g2_len,
           W1, b1, W2, b2):
    n, d = graph1_x.shape
    ed = W1.shape[0]
    p = idx_left.shape[0]
    nseg = g1_len.shape[0]

    grid_n = 16
    row_blk = n // grid_n
    pc = 128
    pr = p // pc
    blk_r = pr // grid_n
    prep = pl.pallas_call(
        _prep_body,
        grid=(grid_n,),
        in_specs=[
            pl.BlockSpec(memory_space=pltpu.SMEM),
            pl.BlockSpec(memory_space=pltpu.SMEM),
            pl.BlockSpec((row_blk, d), lambda i: (i, 0)),
            pl.BlockSpec((row_blk, d), lambda i: (i, 0)),
            pl.BlockSpec((d, ed), lambda i: (0, 0)),
            pl.BlockSpec((d, ed), lambda i: (0, 0)),
            pl.BlockSpec((1, ed), lambda i: (0, 0)),
            pl.BlockSpec((blk_r, pc), lambda i: (i, 0)),
            pl.BlockSpec((blk_r, pc), lambda i: (i, 0)),
            pl.BlockSpec((blk_r, pc), lambda i: (i, 0)),
        ],
        out_specs=[
            pl.BlockSpec((row_blk, ed), lambda i: (i, 0)),
            pl.BlockSpec((row_blk, ed), lambda i: (i, 0)),
            pl.BlockSpec((blk_r, pc), lambda i: (i, 0)),
            pl.BlockSpec((blk_r, pc), lambda i: (i, 0)),
        ],
        out_shape=[
            jax.ShapeDtypeStruct((n, ed), jnp.float32),
            jax.ShapeDtypeStruct((n, ed), jnp.float32),
            jax.ShapeDtypeStruct((pr, pc), jnp.int32),
            jax.ShapeDtypeStruct((pr, pc), jnp.int32),
        ],
    )
    a_t, c_t, gl2, gr2 = prep(
        g1_len, g2_len, graph1_x, graph2_x, W1[:d], W1[d:], b1.reshape(1, ed),
        idx_left.reshape(pr, pc), idx_right.reshape(pr, pc),
        pair_seg.reshape(pr, pc))
    gl = gl2.reshape(p)
    gr = gr2.reshape(p)

    ppw = p // _NW
    nchunk = ppw // _CH
    mesh = plsc.VectorSubcoreMesh(core_axis_name="c", subcore_axis_name="s")
    sc_call = pl.kernel(
        functools.partial(_sc_body, nchunk, ppw),
        out_type=jax.ShapeDtypeStruct((p,), jnp.float32),
        mesh=mesh,
        compiler_params=pltpu.CompilerParams(needs_layout_passes=False),
        scratch_types=[
            pltpu.VMEM((_CH,), jnp.int32),
            pltpu.VMEM((_CH,), jnp.int32),
            pltpu.VMEM((_CH, ed), jnp.float32),
            pltpu.VMEM((_CH, ed), jnp.float32),
            pltpu.VMEM((_CH,), jnp.int32),
            pltpu.VMEM((_CH,), jnp.int32),
            pltpu.VMEM((_CH, ed), jnp.float32),
            pltpu.VMEM((_CH, ed), jnp.float32),
            pltpu.VMEM((_CH,), jnp.float32),
            pltpu.VMEM((ed,), jnp.float32),
            pltpu.VMEM((_L,), jnp.float32),
            pltpu.SemaphoreType.DMA,
            pltpu.SemaphoreType.DMA,
            pltpu.SemaphoreType.DMA,
            pltpu.SemaphoreType.DMA,
        ],
    )
    b2vec = jnp.full((_L,), b2[0], dtype=jnp.float32)
    out = sc_call(a_t, c_t, gl, gr, W2.reshape(ed), b2vec)
    return out.reshape(p, 1)
